# baseline (speedup 1.0000x reference)
import functools

import jax
import jax.numpy as jnp
from jax import lax
from jax.experimental import pallas as pl
from jax.experimental.pallas import tpu as pltpu
from jax.experimental.pallas import tpu_sc as plsc

NFIELD = 8
NFEAT = 26
VOCAB = 1000
D = 64
B = 1024
NPAIR = (NFEAT * (NFEAT - 1)) // 2
NTOK = 32
NC, NS = 2, 16
NW = NC * NS
BPW = B // NW
NV = D // 16

_PBASE = [0]
for _i in range(1, NFEAT):
    _PBASE.append(_PBASE[-1] + NFEAT - _i)

_BLOCKS = []
for _ga in range(NFIELD):
    for _gb in range(NFIELD):
        _pairs = [(i, j)
                  for i in range(_ga, NFEAT, NFIELD)
                  for j in range(_gb, NFEAT, NFIELD) if i < j]
        if _pairs:
            _BLOCKS.append((_ga, _gb, tuple(_pairs)))


def _body(idx_hbm, table_hbm, out_hbm, idx_v, rows_v, out_v, gsem0, gsem1):
    gsems = (gsem0, gsem1)
    wid = lax.axis_index("s") * NC + lax.axis_index("c")
    row0 = wid * BPW
    pltpu.sync_copy(idx_hbm.at[pl.ds(row0, BPW)], idx_v)

    def gather(r, s):
        return [pltpu.make_async_copy(
                    table_hbm.at[g].at[idx_v.at[r]],
                    rows_v.at[s, g], gsems[s])
                for g in range(NFIELD)]

    def gather_start(r, s):
        for c in gather(r, s):
            c.start()

    def gather_wait(r, s):
        for c in gather(r, s):
            c.wait()

    def compute_row(r, s):
        for ga, gb, pairs in _BLOCKS:
            cache = {}

            def get(g, f):
                if (g, f) not in cache:
                    cache[(g, f)] = [rows_v[s, g, f, pl.ds(16 * k, 16)]
                                     for k in range(NV)]
                return cache[(g, f)]

            for i, j in pairs:
                a = get(gb, i)
                b = get(ga, j)
                p = _PBASE[i] + j - i - 1
                for k in range(NV):
                    out_v[p, pl.ds(16 * k, 16)] = a[k] * b[k]
        pltpu.sync_copy(out_v, out_hbm.at[row0 + r])

    gather_start(0, 0)
    nit = BPW // 2

    def two_rows(it, carry):
        r0 = 2 * it
        gather_start(r0 + 1, 1)
        gather_wait(r0, 0)
        compute_row(r0, 0)
        gather_start(jnp.minimum(r0 + 2, BPW - 1), 0)
        gather_wait(r0 + 1, 1)
        compute_row(r0 + 1, 1)
        return carry

    lax.fori_loop(0, nit, two_rows, 0)
    gather_wait(BPW - 1, 0)


def kernel(input_x, W):
    token = input_x[0].astype(jnp.int32)
    f_off = jnp.arange(NFEAT, dtype=jnp.int32) * VOCAB
    idx = token + f_off[None, :]
    idx = jnp.pad(idx, ((0, 0), (0, NTOK - NFEAT)))

    run = pl.kernel(
        _body,
        out_type=jax.ShapeDtypeStruct((B, NPAIR, D), jnp.float32),
        mesh=plsc.VectorSubcoreMesh(
            core_axis_name="c", subcore_axis_name="s",
            num_cores=NC, num_subcores=NS),
        scratch_types=[
            pltpu.VMEM((BPW, NTOK), jnp.int32),
            pltpu.VMEM((2, NFIELD, NTOK, D), jnp.float32),
            pltpu.VMEM((NPAIR, D), jnp.float32),
            pltpu.SemaphoreType.DMA,
            pltpu.SemaphoreType.DMA,
        ],
        compiler_params=pltpu.CompilerParams(use_tc_tiling_on_sc=False),
    )
    return run(idx, W)


# speedup vs baseline: 1.3853x; 1.3853x over previous
import functools

import jax
import jax.numpy as jnp
from jax import lax
from jax.experimental import pallas as pl
from jax.experimental.pallas import tpu as pltpu
from jax.experimental.pallas import tpu_sc as plsc

NFIELD = 8
NFEAT = 26
VOCAB = 1000
D = 64
B = 1024
NPAIR = (NFEAT * (NFEAT - 1)) // 2
NROW = NFIELD * NFEAT
NC, NS = 2, 16
NW = NC * NS
BPW = B // NW
HALF = NROW // 2
NV = D // 16

_PBASE = [0]
for _i in range(1, NFEAT):
    _PBASE.append(_PBASE[-1] + NFEAT - _i)

_BLOCKS = []
for _ga in range(NFIELD):
    for _gb in range(NFIELD):
        _pairs = [(i, j)
                  for i in range(_ga, NFEAT, NFIELD)
                  for j in range(_gb, NFEAT, NFIELD) if i < j]
        if _pairs:
            _BLOCKS.append((_ga, _gb, tuple(_pairs)))


def _body(idx_hbm, table_hbm, out_hbm, idx_v, rows_v, out_v, gsem0, gsem1):
    gsems = (gsem0, gsem1)
    wid = lax.axis_index("s") * NC + lax.axis_index("c")
    row0 = wid * BPW
    pltpu.sync_copy(idx_hbm.at[pl.ds(row0 * NROW, BPW * NROW)], idx_v)

    def gather(r, s):
        return [pltpu.make_async_copy(
                    table_hbm.at[idx_v.at[pl.ds(r * NROW + h * HALF, HALF)]],
                    rows_v.at[s, pl.ds(h * HALF, HALF)], gsems[s])
                for h in range(2)]

    def gather_start(r, s):
        for c in gather(r, s):
            c.start()

    def gather_wait(r, s):
        for c in gather(r, s):
            c.wait()

    def compute_row(r, s):
        for ga, gb, pairs in _BLOCKS:
            cache = {}

            def get(row):
                if row not in cache:
                    cache[row] = [rows_v[s, row, pl.ds(16 * k, 16)]
                                  for k in range(NV)]
                return cache[row]

            for i, j in pairs:
                a = get(gb * NFEAT + i)
                b = get(ga * NFEAT + j)
                p = _PBASE[i] + j - i - 1
                for k in range(NV):
                    out_v[p, pl.ds(16 * k, 16)] = a[k] * b[k]
        pltpu.sync_copy(out_v, out_hbm.at[row0 + r])

    gather_start(0, 0)
    nit = BPW // 2

    def two_rows(it, carry):
        r0 = 2 * it
        gather_start(r0 + 1, 1)
        gather_wait(r0, 0)
        compute_row(r0, 0)
        gather_start(jnp.minimum(r0 + 2, BPW - 1), 0)
        gather_wait(r0 + 1, 1)
        compute_row(r0 + 1, 1)
        return carry

    lax.fori_loop(0, nit, two_rows, 0)
    gather_wait(BPW - 1, 0)


def kernel(input_x, W):
    token = input_x[0].astype(jnp.int32)
    f_off = jnp.arange(NFEAT, dtype=jnp.int32) * VOCAB
    g_off = jnp.arange(NFIELD, dtype=jnp.int32) * (NFEAT * VOCAB)
    idx = token[:, None, :] + f_off[None, None, :] + g_off[None, :, None]
    idx = idx.reshape(B * NROW)
    table = W.reshape(NFIELD * NFEAT * VOCAB, D)

    run = pl.kernel(
        _body,
        out_type=jax.ShapeDtypeStruct((B, NPAIR, D), jnp.float32),
        mesh=plsc.VectorSubcoreMesh(
            core_axis_name="c", subcore_axis_name="s",
            num_cores=NC, num_subcores=NS),
        scratch_types=[
            pltpu.VMEM((BPW * NROW,), jnp.int32),
            pltpu.VMEM((2, NROW, D), jnp.float32),
            pltpu.VMEM((NPAIR, D), jnp.float32),
            pltpu.SemaphoreType.DMA,
            pltpu.SemaphoreType.DMA,
        ],
        compiler_params=pltpu.CompilerParams(use_tc_tiling_on_sc=False),
    )
    return run(idx, table)


# speedup vs baseline: 1.4546x; 1.0500x over previous
import functools

import jax
import jax.numpy as jnp
from jax import lax
from jax.experimental import pallas as pl
from jax.experimental.pallas import tpu as pltpu
from jax.experimental.pallas import tpu_sc as plsc

NFIELD = 8
NFEAT = 26
VOCAB = 1000
D = 64
B = 1024
NPAIR = (NFEAT * (NFEAT - 1)) // 2
NROW = NFIELD * NFEAT
NC, NS = 2, 16
NW = NC * NS
BPW = B // NW
HALF = NROW // 2
NV = D // 16

_PBASE = [0]
for _i in range(1, NFEAT):
    _PBASE.append(_PBASE[-1] + NFEAT - _i)

_BLOCKS = []
for _ga in range(NFIELD):
    for _gb in range(NFIELD):
        _pairs = [(i, j)
                  for i in range(_ga, NFEAT, NFIELD)
                  for j in range(_gb, NFEAT, NFIELD) if i < j]
        if _pairs:
            _BLOCKS.append((_ga, _gb, tuple(_pairs)))


def _body(idx_hbm, table_hbm, out_hbm, idx_v, rows_v, out_v,
          gsem0, gsem1, osem0, osem1):
    gsems = (gsem0, gsem1)
    osems = (osem0, osem1)
    wid = lax.axis_index("s") * NC + lax.axis_index("c")
    row0 = wid * BPW
    pltpu.sync_copy(idx_hbm.at[pl.ds(row0 * NROW, BPW * NROW)], idx_v)

    def gather(r, s):
        return [pltpu.make_async_copy(
                    table_hbm.at[idx_v.at[pl.ds(r * NROW + h * HALF, HALF)]],
                    rows_v.at[s, pl.ds(h * HALF, HALF)], gsems[s])
                for h in range(2)]

    def gather_start(r, s):
        for c in gather(r, s):
            c.start()

    def gather_wait(r, s):
        for c in gather(r, s):
            c.wait()

    def outwrite(r, s):
        return pltpu.make_async_copy(
            out_v.at[s], out_hbm.at[row0 + r], osems[s])

    def compute_row(r, s):
        for ga, gb, pairs in _BLOCKS:
            cache = {}

            def get(row):
                if row not in cache:
                    cache[row] = [rows_v[s, row, pl.ds(16 * k, 16)]
                                  for k in range(NV)]
                return cache[row]

            for i, j in pairs:
                a = get(gb * NFEAT + i)
                b = get(ga * NFEAT + j)
                p = _PBASE[i] + j - i - 1
                for k in range(NV):
                    out_v[s, p, pl.ds(16 * k, 16)] = a[k] * b[k]

    gather_start(0, 0)
    nit = BPW // 2

    def two_rows(it, carry):
        r0 = 2 * it
        gather_start(r0 + 1, 1)
        gather_wait(r0, 0)

        @pl.when(it >= 1)
        def _():
            outwrite(r0 - 2, 0).wait()
        compute_row(r0, 0)
        outwrite(r0, 0).start()

        gather_start(jnp.minimum(r0 + 2, BPW - 1), 0)
        gather_wait(r0 + 1, 1)

        @pl.when(it >= 1)
        def _():
            outwrite(r0 - 1, 1).wait()
        compute_row(r0 + 1, 1)
        outwrite(r0 + 1, 1).start()
        return carry

    lax.fori_loop(0, nit, two_rows, 0)
    gather_wait(BPW - 1, 0)
    outwrite(BPW - 2, 0).wait()
    outwrite(BPW - 1, 1).wait()


def kernel(input_x, W):
    token = input_x[0].astype(jnp.int32)
    f_off = jnp.arange(NFEAT, dtype=jnp.int32) * VOCAB
    g_off = jnp.arange(NFIELD, dtype=jnp.int32) * (NFEAT * VOCAB)
    idx = token[:, None, :] + f_off[None, None, :] + g_off[None, :, None]
    idx = idx.reshape(B * NROW)
    table = W.reshape(NFIELD * NFEAT * VOCAB, D)

    run = pl.kernel(
        _body,
        out_type=jax.ShapeDtypeStruct((B, NPAIR, D), jnp.float32),
        mesh=plsc.VectorSubcoreMesh(
            core_axis_name="c", subcore_axis_name="s",
            num_cores=NC, num_subcores=NS),
        scratch_types=[
            pltpu.VMEM((BPW * NROW,), jnp.int32),
            pltpu.VMEM((2, NROW, D), jnp.float32),
            pltpu.VMEM((2, NPAIR, D), jnp.float32),
            pltpu.SemaphoreType.DMA,
            pltpu.SemaphoreType.DMA,
            pltpu.SemaphoreType.DMA,
            pltpu.SemaphoreType.DMA,
        ],
        compiler_params=pltpu.CompilerParams(use_tc_tiling_on_sc=False),
    )
    return run(idx, table)
